# Initial kernel scaffold; baseline (speedup 1.0000x reference)
import jax
import jax.numpy as jnp
from jax.experimental import pallas as pl


def kernel(selected_input_acts, input_idx, k_process, process_weights, process_outputs):
    raise NotImplementedError("write your pallas kernel here")


# speedup vs baseline: 2.6351x; 2.6351x over previous
import functools
import math

import jax
import jax.numpy as jnp
from jax import lax
from jax.experimental import pallas as pl
from jax.experimental.pallas import tpu as pltpu

_K_SEL = 8
_S_BLK = 1024


def _gelu_exact(x):
    return 0.5 * x * (1.0 + lax.erf(x * (1.0 / math.sqrt(2.0))))


def _scores_body(idx_ref, acts_ref, w_ref, scores_ref, spw_ref, *, s_total):
    si = pl.program_id(1)
    n_in = w_ref.shape[1]
    io = lax.broadcasted_iota(jnp.int32, (n_in, idx_ref.shape[2]), 0)
    oh = (io == idx_ref[0]).astype(jnp.float32)
    spw = jax.lax.dot(w_ref[...], oh)

    @pl.when(si == 0)
    def _():
        spw_ref[0] = spw

    x = acts_ref[0]
    pa = lax.dot_general(x, spw, (((1,), (1,)), ((), ())))
    part = jnp.sum(_gelu_exact(pa), axis=0, keepdims=True) * (1.0 / s_total)

    @pl.when(si == 0)
    def _():
        scores_ref[0] = part

    @pl.when(si != 0)
    def _():
        scores_ref[0] = scores_ref[0] + part


def _route_body(scores_ref, spw_ref, po_ref, spw_sel_ref, po_sel_ref):
    s = scores_ref[0]
    n_proc = s.shape[1]
    iota_p = lax.broadcasted_iota(jnp.int32, (1, n_proc), 1)
    rows = []
    for _ in range(_K_SEL):
        m = jnp.max(s)
        idx = jnp.min(jnp.where(s == m, iota_p, n_proc))
        hit = iota_p == idx
        rows.append(hit.astype(jnp.float32))
        s = jnp.where(hit, -jnp.inf, s)
    onehot = jnp.concatenate(rows, axis=0)
    spw_sel_ref[0] = jax.lax.dot(onehot, spw_ref[0])
    po_sel_ref[0] = jax.lax.dot(onehot, po_ref[...])


def _out_body(acts_ref, spw_sel_ref, po_sel_ref, out_ref):
    x = acts_ref[0]
    w = spw_sel_ref[0]
    a = _gelu_exact(lax.dot_general(x, w, (((1,), (1,)), ((), ()))))
    out_ref[0] = jax.lax.dot(a, po_sel_ref[0])


def kernel(selected_input_acts, input_idx, k_process, process_weights, process_outputs):
    del k_process
    B, S, k_in = selected_input_acts.shape
    n_proc, n_in = process_weights.shape
    d_model = process_outputs.shape[1]
    n_s = S // _S_BLK

    idx3 = input_idx.reshape(B, 1, k_in)

    scores, spw = pl.pallas_call(
        functools.partial(_scores_body, s_total=S),
        grid=(B, n_s),
        in_specs=[
            pl.BlockSpec((1, 1, k_in), lambda b, s: (b, 0, 0)),
            pl.BlockSpec((1, _S_BLK, k_in), lambda b, s: (b, s, 0)),
            pl.BlockSpec((n_proc, n_in), lambda b, s: (0, 0)),
        ],
        out_specs=[
            pl.BlockSpec((1, 1, n_proc), lambda b, s: (b, 0, 0)),
            pl.BlockSpec((1, n_proc, k_in), lambda b, s: (b, 0, 0)),
        ],
        out_shape=[
            jax.ShapeDtypeStruct((B, 1, n_proc), jnp.float32),
            jax.ShapeDtypeStruct((B, n_proc, k_in), jnp.float32),
        ],
    )(idx3, selected_input_acts, process_weights)

    spw_sel, po_sel = pl.pallas_call(
        _route_body,
        grid=(B,),
        in_specs=[
            pl.BlockSpec((1, 1, n_proc), lambda b: (b, 0, 0)),
            pl.BlockSpec((1, n_proc, k_in), lambda b: (b, 0, 0)),
            pl.BlockSpec((n_proc, d_model), lambda b: (0, 0)),
        ],
        out_specs=[
            pl.BlockSpec((1, _K_SEL, k_in), lambda b: (b, 0, 0)),
            pl.BlockSpec((1, _K_SEL, d_model), lambda b: (b, 0, 0)),
        ],
        out_shape=[
            jax.ShapeDtypeStruct((B, _K_SEL, k_in), jnp.float32),
            jax.ShapeDtypeStruct((B, _K_SEL, d_model), jnp.float32),
        ],
    )(scores, spw, process_outputs)

    out = pl.pallas_call(
        _out_body,
        grid=(B, n_s),
        in_specs=[
            pl.BlockSpec((1, _S_BLK, k_in), lambda b, s: (b, s, 0)),
            pl.BlockSpec((1, _K_SEL, k_in), lambda b, s: (b, 0, 0)),
            pl.BlockSpec((1, _K_SEL, d_model), lambda b, s: (b, 0, 0)),
        ],
        out_specs=pl.BlockSpec((1, _S_BLK, d_model), lambda b, s: (b, s, 0)),
        out_shape=jax.ShapeDtypeStruct((B, S, d_model), jnp.float32),
    )(selected_input_acts, spw_sel, po_sel)
    return out


# speedup vs baseline: 2.8423x; 1.0786x over previous
import functools
import math

import jax
import jax.numpy as jnp
from jax import lax
from jax.experimental import pallas as pl

_K_SEL = 8
_S_BLK1 = 2048
_S_BLK3 = 2048


def _gelu_exact(x):
    return 0.5 * x * (1.0 + lax.erf(x * (1.0 / math.sqrt(2.0))))


def _scores_body(idx_ref, acts_ref, w_ref, scores_ref, spw_ref, *, s_total):
    si = pl.program_id(1)
    n_in = w_ref.shape[1]
    io = lax.broadcasted_iota(jnp.int32, (n_in, idx_ref.shape[2]), 0)
    oh = (io == idx_ref[0]).astype(jnp.float32)
    spw = jax.lax.dot(w_ref[...], oh)

    @pl.when(si == 0)
    def _():
        spw_ref[0] = spw

    x = acts_ref[0]
    pa = lax.dot_general(x, spw, (((1,), (1,)), ((), ())))
    g = _gelu_exact(pa)
    ones = jnp.full((1, g.shape[0]), 1.0 / s_total, jnp.float32)
    part = jax.lax.dot(ones, g)

    @pl.when(si == 0)
    def _():
        scores_ref[0] = part

    @pl.when(si != 0)
    def _():
        scores_ref[0] = scores_ref[0] + part


def _route_body(scores_ref, spw_ref, po_ref, spw_sel_ref, po_sel_ref):
    s = scores_ref[...]
    bq, n_proc = s.shape
    iota_p = lax.broadcasted_iota(jnp.int32, (bq, n_proc), 1)
    rows = []
    for _ in range(_K_SEL):
        m = jnp.max(s, axis=1, keepdims=True)
        idx = jnp.min(jnp.where(s == m, iota_p, n_proc), axis=1, keepdims=True)
        hit = iota_p == idx
        rows.append(hit.astype(jnp.float32))
        s = jnp.where(hit, -jnp.inf, s)
    oh = jnp.stack(rows, axis=1)
    ps = jax.lax.dot(oh.reshape(bq * _K_SEL, n_proc), po_ref[...])
    po_sel_ref[...] = ps.reshape(bq, _K_SEL, po_ref.shape[1])
    for b in range(bq):
        spw_sel_ref[b] = jax.lax.dot(oh[b], spw_ref[b])


def _out_body(acts_ref, spw_sel_ref, po_sel_ref, out_ref):
    x = acts_ref[0]
    w = spw_sel_ref[0]
    a = _gelu_exact(lax.dot_general(x, w, (((1,), (1,)), ((), ()))))
    out_ref[0] = jax.lax.dot(a, po_sel_ref[0])


def kernel(selected_input_acts, input_idx, k_process, process_weights, process_outputs):
    del k_process
    B, S, k_in = selected_input_acts.shape
    n_proc, n_in = process_weights.shape
    d_model = process_outputs.shape[1]

    idx3 = input_idx.reshape(B, 1, k_in)

    scores, spw = pl.pallas_call(
        functools.partial(_scores_body, s_total=S),
        grid=(B, S // _S_BLK1),
        in_specs=[
            pl.BlockSpec((1, 1, k_in), lambda b, s: (b, 0, 0)),
            pl.BlockSpec((1, _S_BLK1, k_in), lambda b, s: (b, s, 0)),
            pl.BlockSpec((n_proc, n_in), lambda b, s: (0, 0)),
        ],
        out_specs=[
            pl.BlockSpec((1, 1, n_proc), lambda b, s: (b, 0, 0)),
            pl.BlockSpec((1, n_proc, k_in), lambda b, s: (b, 0, 0)),
        ],
        out_shape=[
            jax.ShapeDtypeStruct((B, 1, n_proc), jnp.float32),
            jax.ShapeDtypeStruct((B, n_proc, k_in), jnp.float32),
        ],
    )(idx3, selected_input_acts, process_weights)

    spw_sel, po_sel = pl.pallas_call(
        _route_body,
        grid=(1,),
        in_specs=[
            pl.BlockSpec((B, n_proc), lambda i: (0, 0)),
            pl.BlockSpec((B, n_proc, k_in), lambda i: (0, 0, 0)),
            pl.BlockSpec((n_proc, d_model), lambda i: (0, 0)),
        ],
        out_specs=[
            pl.BlockSpec((B, _K_SEL, k_in), lambda i: (0, 0, 0)),
            pl.BlockSpec((B, _K_SEL, d_model), lambda i: (0, 0, 0)),
        ],
        out_shape=[
            jax.ShapeDtypeStruct((B, _K_SEL, k_in), jnp.float32),
            jax.ShapeDtypeStruct((B, _K_SEL, d_model), jnp.float32),
        ],
    )(scores.reshape(B, n_proc), spw, process_outputs)

    out = pl.pallas_call(
        _out_body,
        grid=(B, S // _S_BLK3),
        in_specs=[
            pl.BlockSpec((1, _S_BLK3, k_in), lambda b, s: (b, s, 0)),
            pl.BlockSpec((1, _K_SEL, k_in), lambda b, s: (b, 0, 0)),
            pl.BlockSpec((1, _K_SEL, d_model), lambda b, s: (b, 0, 0)),
        ],
        out_specs=pl.BlockSpec((1, _S_BLK3, d_model), lambda b, s: (b, s, 0)),
        out_shape=jax.ShapeDtypeStruct((B, S, d_model), jnp.float32),
    )(selected_input_acts, spw_sel, po_sel)
    return out


# speedup vs baseline: 2.8624x; 1.0071x over previous
import functools
import math

import jax
import jax.numpy as jnp
from jax import lax
from jax.experimental import pallas as pl

_K_SEL = 8
_S_BLK1 = 2048
_S_BLK3 = 2048


def _gelu_exact(x):
    return 0.5 * x * (1.0 + lax.erf(x * (1.0 / math.sqrt(2.0))))


def _scores_body(idx_ref, acts_ref, w_ref, scores_ref, spw_ref, *, s_total):
    si = pl.program_id(1)
    n_in = w_ref.shape[1]
    io = lax.broadcasted_iota(jnp.int32, (n_in, idx_ref.shape[2]), 0)
    oh = (io == idx_ref[0]).astype(jnp.float32)
    spw = jax.lax.dot(w_ref[...], oh)

    @pl.when(si == 0)
    def _():
        spw_ref[0] = spw

    x = acts_ref[0]
    pa = lax.dot_general(x, spw, (((1,), (1,)), ((), ())))
    u = pa * lax.erf(pa * (1.0 / math.sqrt(2.0)))
    ones = jnp.full((1, x.shape[0]), 0.5 / s_total, jnp.float32)
    colsum = jax.lax.dot(ones, x)
    part = (jax.lax.dot(ones, u)
            + lax.dot_general(colsum, spw, (((1,), (1,)), ((), ()))))

    @pl.when(si == 0)
    def _():
        scores_ref[0] = part

    @pl.when(si != 0)
    def _():
        scores_ref[0] = scores_ref[0] + part


def _route_body(scores_ref, spw_ref, po_ref, spw_sel_ref, po_sel_ref):
    s = scores_ref[...]
    bq, n_proc = s.shape
    iota_p = lax.broadcasted_iota(jnp.int32, (bq, n_proc), 1)
    rows = []
    for _ in range(_K_SEL):
        m = jnp.max(s, axis=1, keepdims=True)
        idx = jnp.min(jnp.where(s == m, iota_p, n_proc), axis=1, keepdims=True)
        hit = iota_p == idx
        rows.append(hit.astype(jnp.float32))
        s = jnp.where(hit, -jnp.inf, s)
    oh = jnp.stack(rows, axis=1)
    ps = jax.lax.dot(oh.reshape(bq * _K_SEL, n_proc), po_ref[...])
    po_sel_ref[...] = ps.reshape(bq, _K_SEL, po_ref.shape[1])
    for b in range(bq):
        spw_sel_ref[b] = jax.lax.dot(oh[b], spw_ref[b])


def _out_body(acts_ref, spw_sel_ref, po_sel_ref, out_ref):
    x = acts_ref[0]
    w = spw_sel_ref[0]
    a = _gelu_exact(lax.dot_general(x, w, (((1,), (1,)), ((), ()))))
    out_ref[0] = jax.lax.dot(a, po_sel_ref[0])


def kernel(selected_input_acts, input_idx, k_process, process_weights, process_outputs):
    del k_process
    B, S, k_in = selected_input_acts.shape
    n_proc, n_in = process_weights.shape
    d_model = process_outputs.shape[1]

    idx3 = input_idx.reshape(B, 1, k_in)

    scores, spw = pl.pallas_call(
        functools.partial(_scores_body, s_total=S),
        grid=(B, S // _S_BLK1),
        in_specs=[
            pl.BlockSpec((1, 1, k_in), lambda b, s: (b, 0, 0)),
            pl.BlockSpec((1, _S_BLK1, k_in), lambda b, s: (b, s, 0)),
            pl.BlockSpec((n_proc, n_in), lambda b, s: (0, 0)),
        ],
        out_specs=[
            pl.BlockSpec((1, 1, n_proc), lambda b, s: (b, 0, 0)),
            pl.BlockSpec((1, n_proc, k_in), lambda b, s: (b, 0, 0)),
        ],
        out_shape=[
            jax.ShapeDtypeStruct((B, 1, n_proc), jnp.float32),
            jax.ShapeDtypeStruct((B, n_proc, k_in), jnp.float32),
        ],
    )(idx3, selected_input_acts, process_weights)

    spw_sel, po_sel = pl.pallas_call(
        _route_body,
        grid=(1,),
        in_specs=[
            pl.BlockSpec((B, n_proc), lambda i: (0, 0)),
            pl.BlockSpec((B, n_proc, k_in), lambda i: (0, 0, 0)),
            pl.BlockSpec((n_proc, d_model), lambda i: (0, 0)),
        ],
        out_specs=[
            pl.BlockSpec((B, _K_SEL, k_in), lambda i: (0, 0, 0)),
            pl.BlockSpec((B, _K_SEL, d_model), lambda i: (0, 0, 0)),
        ],
        out_shape=[
            jax.ShapeDtypeStruct((B, _K_SEL, k_in), jnp.float32),
            jax.ShapeDtypeStruct((B, _K_SEL, d_model), jnp.float32),
        ],
    )(scores.reshape(B, n_proc), spw, process_outputs)

    out = pl.pallas_call(
        _out_body,
        grid=(B, S // _S_BLK3),
        in_specs=[
            pl.BlockSpec((1, _S_BLK3, k_in), lambda b, s: (b, s, 0)),
            pl.BlockSpec((1, _K_SEL, k_in), lambda b, s: (b, 0, 0)),
            pl.BlockSpec((1, _K_SEL, d_model), lambda b, s: (b, 0, 0)),
        ],
        out_specs=pl.BlockSpec((1, _S_BLK3, d_model), lambda b, s: (b, s, 0)),
        out_shape=jax.ShapeDtypeStruct((B, S, d_model), jnp.float32),
    )(selected_input_acts, spw_sel, po_sel)
    return out
